# baseline (speedup 1.0000x reference)
import functools

import jax
import jax.numpy as jnp
from jax import lax
from jax.experimental import pallas as pl
from jax.experimental.pallas import tpu as pltpu
from jax.experimental.pallas import tpu_sc as plsc

N_BITS = 20
OUT_DIM = 16
BATCH = 16384

NUM_CORES = 2
NUM_SUBCORES = 16
LANES = 16
NUM_WORKERS = NUM_CORES * NUM_SUBCORES
BPW = BATCH // NUM_WORKERS
CHUNK = 128
NCHUNK = BPW // CHUNK
GROUPS_PER_CHUNK = CHUNK // LANES


def _lookup_body(img_hbm, enc_hbm, out_hbm, img_v, idx_v, scale_v, rows_v, sem):
    wid = lax.axis_index("s") * NUM_CORES + lax.axis_index("c")
    base = wid * BPW

    pltpu.sync_copy(img_hbm.at[pl.ds(base * N_BITS, BPW * N_BITS)], img_v)

    lanes = jnp.arange(LANES, dtype=jnp.int32)
    copies = []
    for c in range(NCHUNK):
        def grp(g, carry, c=c):
            row0 = c * CHUNK + g * LANES
            flat0 = (row0 + lanes) * N_BITS
            acc = jnp.zeros((LANES,), jnp.int32)
            sab = jnp.zeros((LANES,), jnp.float32)
            for n in range(N_BITS):
                v = plsc.load_gather(img_v, [flat0 + n])
                acc = acc + jnp.where(v > 0, jnp.int32(1 << n), jnp.int32(0))
                sab = sab + jnp.abs(v)
            idx_v[c, pl.ds(g * LANES, LANES)] = acc
            scale_v[pl.ds(row0, LANES)] = sab * (1.0 / N_BITS)
            return carry
        lax.fori_loop(0, GROUPS_PER_CHUNK, grp, 0)
        copies.append(
            pltpu.async_copy(
                enc_hbm.at[idx_v.at[c]],
                rows_v.at[pl.ds(c * CHUNK, CHUNK)],
                sem,
            )
        )
    for cp in copies:
        cp.wait()

    def mul_row(j, carry):
        s = plsc.load_gather(scale_v, [jnp.broadcast_to(j, (LANES,))])
        rows_v[j, :] = rows_v[j, :] * s
        return carry
    lax.fori_loop(0, BPW, mul_row, 0)

    pltpu.sync_copy(rows_v, out_hbm.at[pl.ds(base, BPW), :])


_lookup = functools.partial(
    pl.kernel,
    out_type=jax.ShapeDtypeStruct((BATCH, OUT_DIM), jnp.float32),
    mesh=plsc.VectorSubcoreMesh(core_axis_name="c", subcore_axis_name="s"),
    compiler_params=pltpu.CompilerParams(
        needs_layout_passes=False, use_tc_tiling_on_sc=False
    ),
    scratch_types=[
        pltpu.VMEM((BPW * N_BITS,), jnp.float32),
        pltpu.VMEM((NCHUNK, CHUNK), jnp.int32),
        pltpu.VMEM((BPW,), jnp.float32),
        pltpu.VMEM((BPW, OUT_DIM), jnp.float32),
        pltpu.SemaphoreType.DMA,
    ],
)(_lookup_body)


def kernel(image, encoding):
    return _lookup(image.reshape(-1), encoding)


# speedup vs baseline: 8.4568x; 8.4568x over previous
import functools

import jax
import jax.numpy as jnp
from jax import lax
from jax.experimental import pallas as pl
from jax.experimental.pallas import tpu as pltpu
from jax.experimental.pallas import tpu_sc as plsc

N_BITS = 20
OUT_DIM = 16
BATCH = 16384
NROWS = 1 << N_BITS

NUM_CORES = 2
NUM_SUBCORES = 16
LANES = 16
NUM_WORKERS = NUM_CORES * NUM_SUBCORES
BPW = BATCH // NUM_WORKERS
BLK = 128
NBLK = BPW // BLK
GRP = BLK // LANES
HALF_WORDS = (NROWS // 128) * 1024


def _lookup_body(img_hbm, enc_hbm, out_hbm, img_v, idx_v, scale_v, outt_v, sem):
    wid = lax.axis_index("s") * NUM_CORES + lax.axis_index("c")
    base = wid * BPW

    pltpu.sync_copy(img_hbm.at[pl.ds(base * N_BITS, BPW * N_BITS)], img_v)

    lanes = jnp.arange(LANES, dtype=jnp.int32)
    copies = []
    for blk in range(NBLK):
        def grp_fn(g, carry, blk=blk):
            b0 = blk * BLK + g * LANES
            flat0 = (b0 + lanes) * N_BITS
            acc = jnp.zeros((LANES,), jnp.int32)
            sab = jnp.zeros((LANES,), jnp.float32)
            for n in range(N_BITS):
                v = plsc.load_gather(img_v, [flat0 + n])
                acc = acc + jnp.where(v > 0, jnp.int32(1 << n), jnp.int32(0))
                sab = sab + jnp.abs(v)
            scale_v[pl.ds(b0, LANES)] = sab * (1.0 / N_BITS)
            common = ((acc >> 7) << 10) + (acc & 127)
            for d in range(OUT_DIM):
                i, r = d // 8, d % 8
                idx_v[i * 32 + blk * 8 + r, pl.ds(g * LANES, LANES)] = (
                    common + (i * HALF_WORDS + r * 128)
                )
            return carry
        lax.fori_loop(0, GRP, grp_fn, 0)
        for d in range(OUT_DIM):
            i, r = d // 8, d % 8
            row = i * 32 + blk * 8 + r
            copies.append(
                pltpu.async_copy(
                    enc_hbm.at[idx_v.at[row]],
                    outt_v.at[pl.ds(row * BLK, BLK)],
                    sem,
                )
            )
    for cp in copies:
        cp.wait()

    for i in range(2):
        for blk in range(NBLK):
            for r in range(8):
                k0 = (i * 32 + blk * 8 + r) * BLK
                def mul_fn(g, carry, k0=k0, blk=blk):
                    sl = pl.ds(k0 + g * LANES, LANES)
                    outt_v[sl] = outt_v[sl] * scale_v[pl.ds(blk * BLK + g * LANES, LANES)]
                    return carry
                lax.fori_loop(0, GRP, mul_fn, 0)

    for i in range(2):
        for blk in range(NBLK):
            src = outt_v.at[pl.ds((i * 32 + blk * 8) * BLK, 8 * BLK)]
            dst0 = (i * (BATCH // 128) + wid * NBLK + blk) * 1024
            pltpu.sync_copy(src, out_hbm.at[pl.ds(dst0, 8 * BLK)])


_lookup = functools.partial(
    pl.kernel,
    out_type=jax.ShapeDtypeStruct((2 * (BATCH // 128) * 1024,), jnp.float32),
    mesh=plsc.VectorSubcoreMesh(core_axis_name="c", subcore_axis_name="s"),
    compiler_params=pltpu.CompilerParams(
        needs_layout_passes=False, use_tc_tiling_on_sc=False
    ),
    scratch_types=[
        pltpu.VMEM((BPW * N_BITS,), jnp.float32),
        pltpu.VMEM((64, BLK), jnp.int32),
        pltpu.VMEM((BPW,), jnp.float32),
        pltpu.VMEM((64 * BLK,), jnp.float32),
        pltpu.SemaphoreType.DMA,
    ],
)(_lookup_body)


def kernel(image, encoding):
    enc_flat = (
        encoding.reshape(NROWS // 128, 128, 2, 8)
        .transpose(2, 0, 3, 1)
        .reshape(-1)
    )
    outt = _lookup(image.reshape(-1), enc_flat)
    return (
        outt.reshape(2, BATCH // 128, 8, 128)
        .transpose(1, 3, 0, 2)
        .reshape(BATCH, OUT_DIM)
    )


# speedup vs baseline: 12.3037x; 1.4549x over previous
import functools

import jax
import jax.numpy as jnp
from jax import lax
from jax.experimental import pallas as pl
from jax.experimental.pallas import tpu as pltpu
from jax.experimental.pallas import tpu_sc as plsc

N_BITS = 20
OUT_DIM = 16
BATCH = 16384
NROWS = 1 << N_BITS

NUM_CORES = 2
NUM_SUBCORES = 16
LANES = 16
NUM_WORKERS = NUM_CORES * NUM_SUBCORES
BPW = BATCH // NUM_WORKERS
BLK = 128
NBLK = BPW // BLK
GRP = BLK // LANES
HALF_WORDS = (NROWS // 128) * 1024


def _code_body(imgt_ref, v_ref, s_ref):
    a = imgt_ref[...]
    pw = jnp.left_shift(
        jnp.int32(1), lax.broadcasted_iota(jnp.int32, (N_BITS, 1), 0)
    )
    v_ref[...] = jnp.sum(jnp.where(a > 0, pw, jnp.int32(0)), axis=0)
    s_ref[...] = jnp.sum(jnp.abs(a), axis=0) * (1.0 / N_BITS)


_code = pl.pallas_call(
    _code_body,
    out_shape=[
        jax.ShapeDtypeStruct((BATCH,), jnp.int32),
        jax.ShapeDtypeStruct((BATCH,), jnp.float32),
    ],
)


def _gather_body(
    v_hbm, scale_hbm, enc_hbm, out_hbm,
    v_v, scale_v, idx_v, outt_v, sem0, sem1, sem2, sem3, wsem,
):
    sems = (sem0, sem1, sem2, sem3)
    wid = lax.axis_index("s") * NUM_CORES + lax.axis_index("c")
    base = wid * BPW

    pltpu.sync_copy(v_hbm.at[pl.ds(base, BPW)], v_v)
    pltpu.sync_copy(scale_hbm.at[pl.ds(base, BPW)], scale_v)

    copies = []
    for blk in range(NBLK):
        def build_fn(g, carry, blk=blk):
            vv = v_v[pl.ds(blk * BLK + g * LANES, LANES)]
            common = ((vv >> 7) << 10) + (vv & 127)
            for d in range(OUT_DIM):
                i, r = d // 8, d % 8
                idx_v[i * 32 + blk * 8 + r, pl.ds(g * LANES, LANES)] = (
                    common + (i * HALF_WORDS + r * 128)
                )
            return carry
        lax.fori_loop(0, GRP, build_fn, 0)
        blk_copies = []
        for d in range(OUT_DIM):
            i, r = d // 8, d % 8
            row = i * 32 + blk * 8 + r
            blk_copies.append(
                pltpu.async_copy(
                    enc_hbm.at[idx_v.at[row]],
                    outt_v.at[pl.ds(row * BLK, BLK)],
                    sems[blk],
                )
            )
        copies.append(blk_copies)

    wcopies = []
    for blk in range(NBLK):
        for cp in copies[blk]:
            cp.wait()
        for d in range(OUT_DIM):
            i, r = d // 8, d % 8
            k0 = (i * 32 + blk * 8 + r) * BLK
            def mul_fn(g, carry, k0=k0, blk=blk):
                sl = pl.ds(k0 + g * LANES, LANES)
                outt_v[sl] = outt_v[sl] * scale_v[pl.ds(blk * BLK + g * LANES, LANES)]
                return carry
            lax.fori_loop(0, GRP, mul_fn, 0)
        for i in range(2):
            src = outt_v.at[pl.ds((i * 32 + blk * 8) * BLK, 8 * BLK)]
            dst0 = (i * (BATCH // 128) + wid * NBLK + blk) * 1024
            wcopies.append(
                pltpu.async_copy(src, out_hbm.at[pl.ds(dst0, 8 * BLK)], wsem)
            )
    for cp in wcopies:
        cp.wait()


_gather = functools.partial(
    pl.kernel,
    out_type=jax.ShapeDtypeStruct((2 * (BATCH // 128) * 1024,), jnp.float32),
    mesh=plsc.VectorSubcoreMesh(core_axis_name="c", subcore_axis_name="s"),
    compiler_params=pltpu.CompilerParams(
        needs_layout_passes=False, use_tc_tiling_on_sc=False
    ),
    scratch_types=[
        pltpu.VMEM((BPW,), jnp.int32),
        pltpu.VMEM((BPW,), jnp.float32),
        pltpu.VMEM((64, BLK), jnp.int32),
        pltpu.VMEM((64 * BLK,), jnp.float32),
        pltpu.SemaphoreType.DMA,
        pltpu.SemaphoreType.DMA,
        pltpu.SemaphoreType.DMA,
        pltpu.SemaphoreType.DMA,
        pltpu.SemaphoreType.DMA,
    ],
)(_gather_body)


def kernel(image, encoding):
    enc_flat = (
        encoding.reshape(NROWS // 128, 128, 2, 8)
        .transpose(2, 0, 3, 1)
        .reshape(-1)
    )
    v, scale = _code(image.T)
    outt = _gather(v, scale, enc_flat)
    return (
        outt.reshape(2, BATCH // 128, 8, 128)
        .transpose(1, 3, 0, 2)
        .reshape(BATCH, OUT_DIM)
    )


# speedup vs baseline: 12.8339x; 1.0431x over previous
import functools

import jax
import jax.numpy as jnp
from jax import lax
from jax.experimental import pallas as pl
from jax.experimental.pallas import tpu as pltpu
from jax.experimental.pallas import tpu_sc as plsc

N_BITS = 20
OUT_DIM = 16
BATCH = 16384
NROWS = 1 << N_BITS

NUM_CORES = 2
NUM_SUBCORES = 16
LANES = 16
NUM_WORKERS = NUM_CORES * NUM_SUBCORES
BPW = BATCH // NUM_WORKERS
BLK = 128
NBLK = BPW // BLK
GRP = BLK // LANES
HALF_WORDS = (NROWS // 128) * 1024


def _code_body(imgt_ref, v_ref, s_ref):
    a = imgt_ref[...]
    pw = jnp.left_shift(
        jnp.int32(1), lax.broadcasted_iota(jnp.int32, (N_BITS, 1), 0)
    )
    v_ref[...] = jnp.sum(jnp.where(a > 0, pw, jnp.int32(0)), axis=0)
    s_ref[...] = jnp.sum(jnp.abs(a), axis=0) * (1.0 / N_BITS)


_code = pl.pallas_call(
    _code_body,
    out_shape=[
        jax.ShapeDtypeStruct((BATCH,), jnp.int32),
        jax.ShapeDtypeStruct((BATCH,), jnp.float32),
    ],
)


def _gather_body(
    v_hbm, scale_hbm, enc_hbm, out_hbm,
    v_v, scale_v, idx_v, outt_v, sem0, sem1, sem2, sem3, wsem,
):
    sems = (sem0, sem1, sem2, sem3)
    wid = lax.axis_index("s") * NUM_CORES + lax.axis_index("c")
    base = wid * BPW

    pltpu.sync_copy(v_hbm.at[pl.ds(base, BPW)], v_v)
    pltpu.sync_copy(scale_hbm.at[pl.ds(base, BPW)], scale_v)

    for blk in range(NBLK):
        def build_fn(g, carry, blk=blk):
            vv = v_v[pl.ds(blk * BLK + g * LANES, LANES)]
            common = ((vv >> 7) << 10) + (vv & 127)
            for d in range(OUT_DIM):
                i, r = d // 8, d % 8
                idx_v[blk, i * 8 + r, pl.ds(g * LANES, LANES)] = (
                    common + (i * HALF_WORDS + r * 128)
                )
            return carry
        lax.fori_loop(0, GRP, build_fn, 0)

        def fire_fn(d, carry, blk=blk):
            pltpu.async_copy(
                enc_hbm.at[idx_v.at[blk, d]], outt_v.at[blk, d], sems[blk]
            )
            return carry
        lax.fori_loop(0, OUT_DIM, fire_fn, 0)

    wcopies = []
    for blk in range(NBLK):
        def drain_fn(d, carry, blk=blk):
            pltpu.make_async_copy(
                enc_hbm.at[idx_v.at[blk, d]], outt_v.at[blk, d], sems[blk]
            ).wait()
            return carry
        lax.fori_loop(0, OUT_DIM, drain_fn, 0)

        def mul_fn(m, carry, blk=blk):
            d = m // GRP
            g = m - d * GRP
            sl = pl.ds(g * LANES, LANES)
            outt_v[blk, d, sl] = (
                outt_v[blk, d, sl] * scale_v[pl.ds(blk * BLK + g * LANES, LANES)]
            )
            return carry
        lax.fori_loop(0, OUT_DIM * GRP, mul_fn, 0)

        for i in range(2):
            src = outt_v.at[blk, pl.ds(i * 8, 8)]
            tidx = i * (BATCH // 128) + wid * NBLK + blk
            wcopies.append(
                pltpu.async_copy(src, out_hbm.at[tidx], wsem)
            )
    for cp in wcopies:
        cp.wait()


_gather = functools.partial(
    pl.kernel,
    out_type=jax.ShapeDtypeStruct((2 * (BATCH // 128), 8, 128), jnp.float32),
    mesh=plsc.VectorSubcoreMesh(core_axis_name="c", subcore_axis_name="s"),
    compiler_params=pltpu.CompilerParams(
        needs_layout_passes=False, use_tc_tiling_on_sc=False
    ),
    scratch_types=[
        pltpu.VMEM((BPW,), jnp.int32),
        pltpu.VMEM((BPW,), jnp.float32),
        pltpu.VMEM((NBLK, OUT_DIM, BLK), jnp.int32),
        pltpu.VMEM((NBLK, OUT_DIM, BLK), jnp.float32),
        pltpu.SemaphoreType.DMA,
        pltpu.SemaphoreType.DMA,
        pltpu.SemaphoreType.DMA,
        pltpu.SemaphoreType.DMA,
        pltpu.SemaphoreType.DMA,
    ],
)(_gather_body)


def kernel(image, encoding):
    enc_flat = (
        encoding.reshape(NROWS // 128, 128, 2, 8)
        .transpose(2, 0, 3, 1)
        .reshape(-1)
    )
    v, scale = _code(image.T)
    outt = _gather(v, scale, enc_flat)
    return (
        outt.reshape(2, BATCH // 128, 8, 128)
        .transpose(1, 3, 0, 2)
        .reshape(BATCH, OUT_DIM)
    )


# speedup vs baseline: 13.3330x; 1.0389x over previous
import functools

import jax
import jax.numpy as jnp
from jax import lax
from jax.experimental import pallas as pl
from jax.experimental.pallas import tpu as pltpu
from jax.experimental.pallas import tpu_sc as plsc

N_BITS = 20
OUT_DIM = 16
BATCH = 16384
NROWS = 1 << N_BITS

NUM_CORES = 2
NUM_SUBCORES = 16
LANES = 16
NUM_WORKERS = NUM_CORES * NUM_SUBCORES
BPW = BATCH // NUM_WORKERS
BLK = 128
NBLK = BPW // BLK
GRP = BLK // LANES
HALF_WORDS = (NROWS // 128) * 1024


def _code_body(imgt_ref, v_ref, s_ref):
    a = imgt_ref[...]
    pw = jnp.left_shift(
        jnp.int32(1), lax.broadcasted_iota(jnp.int32, (N_BITS, 1), 0)
    )
    v_ref[...] = jnp.sum(jnp.where(a > 0, pw, jnp.int32(0)), axis=0)
    s_ref[...] = jnp.sum(jnp.abs(a), axis=0) * (1.0 / N_BITS)


_code = pl.pallas_call(
    _code_body,
    out_shape=[
        jax.ShapeDtypeStruct((BATCH,), jnp.int32),
        jax.ShapeDtypeStruct((BATCH,), jnp.float32),
    ],
)


def _gather_body(
    v_hbm, scale_hbm, enc_hbm, out_hbm,
    v_v, scale_v, idx_v, outt_v, sem0, sem1, sem2, sem3, wsem,
):
    sems = (sem0, sem1, sem2, sem3)
    wid = lax.axis_index("s") * NUM_CORES + lax.axis_index("c")
    base = wid * BPW

    pltpu.sync_copy(v_hbm.at[pl.ds(base, BPW)], v_v)
    pltpu.sync_copy(scale_hbm.at[pl.ds(base, BPW)], scale_v)

    for blk in range(NBLK):
        def build_fn(g, carry, blk=blk):
            vv = v_v[pl.ds(blk * BLK + g * LANES, LANES)]
            common = ((vv >> 7) << 10) + (vv & 127)
            for d in range(OUT_DIM):
                i, r = d // 8, d % 8
                idx_v[blk, i * 8 + r, pl.ds(g * LANES, LANES)] = (
                    common + (i * HALF_WORDS + r * 128)
                )
            return carry
        lax.fori_loop(0, GRP, build_fn, 0)

        def fire_fn(d, carry, blk=blk):
            pltpu.async_copy(
                enc_hbm.at[idx_v.at[blk, d]], outt_v.at[blk, d], sems[blk]
            )
            return carry
        lax.fori_loop(0, OUT_DIM, fire_fn, 0)

    wcopies = []
    for blk in range(NBLK):
        def drain_fn(d, carry, blk=blk):
            pltpu.make_async_copy(
                enc_hbm.at[idx_v.at[blk, d]], outt_v.at[blk, d], sems[blk]
            ).wait()
            return carry
        lax.fori_loop(0, OUT_DIM, drain_fn, 0)

        svs = [
            scale_v[pl.ds(blk * BLK + g * LANES, LANES)] for g in range(GRP)
        ]

        def mul_fn(d, carry, blk=blk, svs=svs):
            for g in range(GRP):
                sl = pl.ds(g * LANES, LANES)
                outt_v[blk, d, sl] = outt_v[blk, d, sl] * svs[g]
            return carry
        lax.fori_loop(0, OUT_DIM, mul_fn, 0)

        for i in range(2):
            src = outt_v.at[blk, pl.ds(i * 8, 8)]
            tidx = i * (BATCH // 128) + wid * NBLK + blk
            wcopies.append(
                pltpu.async_copy(src, out_hbm.at[tidx], wsem)
            )
    for cp in wcopies:
        cp.wait()


_gather = functools.partial(
    pl.kernel,
    out_type=jax.ShapeDtypeStruct((2 * (BATCH // 128), 8, 128), jnp.float32),
    mesh=plsc.VectorSubcoreMesh(core_axis_name="c", subcore_axis_name="s"),
    compiler_params=pltpu.CompilerParams(
        needs_layout_passes=False, use_tc_tiling_on_sc=False
    ),
    scratch_types=[
        pltpu.VMEM((BPW,), jnp.int32),
        pltpu.VMEM((BPW,), jnp.float32),
        pltpu.VMEM((NBLK, OUT_DIM, BLK), jnp.int32),
        pltpu.VMEM((NBLK, OUT_DIM, BLK), jnp.float32),
        pltpu.SemaphoreType.DMA,
        pltpu.SemaphoreType.DMA,
        pltpu.SemaphoreType.DMA,
        pltpu.SemaphoreType.DMA,
        pltpu.SemaphoreType.DMA,
    ],
)(_gather_body)


def kernel(image, encoding):
    enc_flat = (
        encoding.reshape(NROWS // 128, 128, 2, 8)
        .transpose(2, 0, 3, 1)
        .reshape(-1)
    )
    v, scale = _code(image.T)
    outt = _gather(v, scale, enc_flat)
    return (
        outt.reshape(2, BATCH // 128, 8, 128)
        .transpose(1, 3, 0, 2)
        .reshape(BATCH, OUT_DIM)
    )


# speedup vs baseline: 13.8685x; 1.0402x over previous
import functools

import jax
import jax.numpy as jnp
from jax import lax
from jax.experimental import pallas as pl
from jax.experimental.pallas import tpu as pltpu
from jax.experimental.pallas import tpu_sc as plsc

N_BITS = 20
OUT_DIM = 16
BATCH = 16384
NROWS = 1 << N_BITS

NUM_CORES = 2
NUM_SUBCORES = 16
LANES = 16
NUM_WORKERS = NUM_CORES * NUM_SUBCORES
BPW = BATCH // NUM_WORKERS
BLK = 128
NBLK = BPW // BLK
GRP = BLK // LANES
BLKW = BLK * OUT_DIM
HALF_WORDS = (NROWS // 128) * 1024


def _code_body(imgt_ref, v_ref, s_ref):
    a = imgt_ref[...]
    pw = jnp.left_shift(
        jnp.int32(1), lax.broadcasted_iota(jnp.int32, (N_BITS, 1), 0)
    )
    v_ref[...] = jnp.sum(jnp.where(a > 0, pw, jnp.int32(0)), axis=0)
    s_ref[...] = jnp.sum(jnp.abs(a), axis=0) * (1.0 / N_BITS)


_code = pl.pallas_call(
    _code_body,
    out_shape=[
        jax.ShapeDtypeStruct((BATCH,), jnp.int32),
        jax.ShapeDtypeStruct((BATCH,), jnp.float32),
    ],
)


def _gather_body(
    v_hbm, scale_hbm, enc_hbm, out_hbm,
    v_v, scale_v, idx_v, outt_v, sem0, sem1, sem2, sem3, wsem,
):
    sems = (sem0, sem1, sem2, sem3)
    wid = lax.axis_index("s") * NUM_CORES + lax.axis_index("c")
    base = wid * BPW

    pltpu.sync_copy(v_hbm.at[pl.ds(base, BPW)], v_v)
    pltpu.sync_copy(scale_hbm.at[pl.ds(base, BPW)], scale_v)

    copies = []
    for blk in range(NBLK):
        def build_fn(g, carry, blk=blk):
            vv = v_v[pl.ds(blk * BLK + g * LANES, LANES)]
            common = ((vv >> 7) << 10) + (vv & 127)
            for d in range(OUT_DIM):
                i, r = d // 8, d % 8
                idx_v[blk, pl.ds((i * 8 + r) * BLK + g * LANES, LANES)] = (
                    common + (i * HALF_WORDS + r * 128)
                )
            return carry
        lax.fori_loop(0, GRP, build_fn, 0)
        copies.append(
            pltpu.async_copy(
                enc_hbm.at[idx_v.at[blk]], outt_v.at[blk], sems[blk]
            )
        )

    wcopies = []
    for blk in range(NBLK):
        copies[blk].wait()

        svs = [
            scale_v[pl.ds(blk * BLK + g * LANES, LANES)] for g in range(GRP)
        ]

        def mul_fn(d, carry, blk=blk, svs=svs):
            for g in range(GRP):
                sl = pl.ds(d * BLK + g * LANES, LANES)
                outt_v[blk, sl] = outt_v[blk, sl] * svs[g]
            return carry
        lax.fori_loop(0, OUT_DIM, mul_fn, 0)

        for i in range(2):
            src = outt_v.at[blk, pl.ds(i * 8 * BLK, 8 * BLK)]
            dst0 = (i * (BATCH // 128) + wid * NBLK + blk) * 1024
            wcopies.append(
                pltpu.async_copy(src, out_hbm.at[pl.ds(dst0, 8 * BLK)], wsem)
            )
    for cp in wcopies:
        cp.wait()


_gather = functools.partial(
    pl.kernel,
    out_type=jax.ShapeDtypeStruct((2 * (BATCH // 128) * 1024,), jnp.float32),
    mesh=plsc.VectorSubcoreMesh(core_axis_name="c", subcore_axis_name="s"),
    compiler_params=pltpu.CompilerParams(
        needs_layout_passes=False, use_tc_tiling_on_sc=False
    ),
    scratch_types=[
        pltpu.VMEM((BPW,), jnp.int32),
        pltpu.VMEM((BPW,), jnp.float32),
        pltpu.VMEM((NBLK, BLKW), jnp.int32),
        pltpu.VMEM((NBLK, BLKW), jnp.float32),
        pltpu.SemaphoreType.DMA,
        pltpu.SemaphoreType.DMA,
        pltpu.SemaphoreType.DMA,
        pltpu.SemaphoreType.DMA,
        pltpu.SemaphoreType.DMA,
    ],
)(_gather_body)


def kernel(image, encoding):
    enc_flat = (
        encoding.reshape(NROWS // 128, 128, 2, 8)
        .transpose(2, 0, 3, 1)
        .reshape(-1)
    )
    v, scale = _code(image.T)
    outt = _gather(v, scale, enc_flat)
    return (
        outt.reshape(2, BATCH // 128, 8, 128)
        .transpose(1, 3, 0, 2)
        .reshape(BATCH, OUT_DIM)
    )
